# baseline (speedup 1.0000x reference)
import functools

import jax
import jax.numpy as jnp
from jax import lax
from jax.experimental import pallas as pl
from jax.experimental.pallas import tpu as pltpu
from jax.experimental.pallas import tpu_sc as plsc

N_USERS = 2000
N = 10000
NP = 10240
D = 128
E = 320000
N_HOPS = 3

NC = 2
NS = 16
DH = D // NC
RPT = NP // NS
K = 128
GC = 4
NG = 40
NCH = NG * GC
EPT = NCH * K
E_PAD = NS * EPT
NQ = RPT // K


def _splat(i):
    return jnp.full((16,), i, dtype=jnp.int32)


_GDN = lax.GatherDimensionNumbers(
    offset_dims=(), collapsed_slice_dims=(0,), start_index_map=(0,))


def _bcast_lane(v16, lane):
    return lax.gather(v16, _splat(lane)[:, None], _GDN, (1,),
                      mode=lax.GatherScatterMode.PROMISE_IN_BOUNDS)


def _sc_body(emb2, er, wr5, out2, snxt, sbf, ebuf, wbuf, gbuf, bbuf,
             se, sg, ss):
    c = lax.axis_index("c")
    s = lax.axis_index("s")
    row0 = s * RPT

    def fetch_group(g, slot):
        pltpu.async_copy(er.at[s, g], ebuf.at[slot], se.at[slot])
        pltpu.async_copy(wr5.at[s, g], wbuf.at[slot], se.at[slot])

    def wait_fetch(slot):
        pltpu.make_async_copy(er.at[s, 0], ebuf.at[slot], se.at[slot]).wait()
        pltpu.make_async_copy(wr5.at[s, 0], wbuf.at[slot], se.at[slot]).wait()

    def gather(slot, k, b):
        pltpu.async_copy(sbf.at[ebuf.at[slot, 2 * k]], bbuf.at[b], sg.at[b])

    def wait_gather(slot, k, b):
        pltpu.make_async_copy(sbf.at[ebuf.at[slot, 2 * k]], bbuf.at[b],
                              sg.at[b]).wait()

    def scatter(slot, k, b):
        pltpu.async_copy(gbuf.at[b], snxt.at[ebuf.at[slot, 2 * k + 1]],
                         ss.at[b], add=True)

    def drain_scatter(slot, k, b):
        pltpu.make_async_copy(gbuf.at[b], snxt.at[ebuf.at[slot, 2 * k + 1]],
                              ss.at[b]).wait()

    def scale(slot, k, b):
        def _scale32(g, _):
            for h in range(2):
                w16 = wbuf[slot, k, pl.ds(32 * g + 16 * h, 16)]
                bs = 32 * g + 16 * h
                for e16 in range(16):
                    wbc = _bcast_lane(w16, e16)
                    for q in range(DH // 32):
                        xw = bbuf[b, bs + e16, pl.ds(16 * q, 16)]
                        x = plsc.bitcast(xw, jnp.bfloat16)
                        a0, a1 = plsc.unpack(
                            x, format=plsc.PackFormat.INTERLEAVED)
                        gbuf[b, bs + e16, pl.ds(32 * q, 16)] = a0 * wbc
                        gbuf[b, bs + e16, pl.ds(32 * q + 16, 16)] = a1 * wbc
            return 0

        lax.fori_loop(0, K // 32, _scale32, 0, unroll=2)

    def pack_rows(nrows):
        def _prow(i, _):
            for h in range(DH // 32):
                a = gbuf[0, i, pl.ds(32 * h, 16)]
                b = gbuf[0, i, pl.ds(32 * h + 16, 16)]
                ab = plsc.pack(a, b, format=plsc.PackFormat.INTERLEAVED)
                bbuf[0, i, pl.ds(16 * h, 16)] = plsc.bitcast(ab, jnp.int32)
            return 0

        lax.fori_loop(0, nrows, _prow, 0)

    for q in range(NQ):
        sl = pl.ds(row0 + K * q, K)
        pltpu.sync_copy(emb2.at[c, sl], gbuf.at[0])
        pack_rows(K)
        pltpu.sync_copy(bbuf.at[0], sbf.at[sl])
    plsc.subcore_barrier()

    for hop in range(N_HOPS):

        def _zrow(i, _):
            for q in range(DH // 16):
                gbuf[0, i, pl.ds(16 * q, 16)] = jnp.zeros((16,), jnp.float32)
            return 0

        lax.fori_loop(0, K, _zrow, 0)
        for q in range(NQ):
            pltpu.sync_copy(gbuf.at[0], snxt.at[pl.ds(row0 + K * q, K)])
        plsc.subcore_barrier()

        def process_group(g, slot):
            wait_fetch(slot)

            @pl.when(g > 0)
            def _():
                drain_scatter(slot, 0, 0)
                drain_scatter(slot, 1, 1)

            @pl.when(g < NG - 1)
            def _():
                fetch_group(g + 1, 1 - slot)

            @pl.loop(0, GC, step=2)
            def _chunkpair(k):
                @pl.when(k > 0)
                def _():
                    drain_scatter(slot, 0, 0)
                gather(slot, k, 0)

                @pl.when(k > 0)
                def _():
                    drain_scatter(slot, 1, 1)
                gather(slot, k + 1, 1)
                wait_gather(slot, k, 0)
                scale(slot, k, 0)
                scatter(slot, k, 0)
                wait_gather(slot, k + 1, 1)
                scale(slot, k + 1, 1)
                scatter(slot, k + 1, 1)

        fetch_group(0, 0)

        @pl.loop(0, NG, step=2)
        def _pair(g):
            process_group(g, 0)
            process_group(g + 1, 1)

        drain_scatter(1, GC - 2, 0)
        drain_scatter(1, GC - 1, 1)
        plsc.subcore_barrier()

        for q in range(NQ):
            sl = pl.ds(row0 + K * q, K)
            pltpu.sync_copy(snxt.at[sl], gbuf.at[0])
            if hop == 0:
                pltpu.sync_copy(emb2.at[c, sl], gbuf.at[1])
            else:
                pltpu.sync_copy(out2.at[c, sl], gbuf.at[1])

            def _acc(i, _):
                for q2 in range(DH // 16):
                    ksl = pl.ds(16 * q2, 16)
                    gbuf[1, i, ksl] = gbuf[1, i, ksl] + gbuf[0, i, ksl]
                return 0

            lax.fori_loop(0, K, _acc, 0)
            pltpu.sync_copy(gbuf.at[1], out2.at[c, sl])
            if hop < N_HOPS - 1:
                pack_rows(K)
                pltpu.sync_copy(bbuf.at[0], sbf.at[sl])
        plsc.subcore_barrier()


@functools.partial(
    pl.kernel,
    out_type=jax.ShapeDtypeStruct((NC, NP, DH), jnp.float32),
    mesh=plsc.VectorSubcoreMesh(core_axis_name="c", subcore_axis_name="s"),
    compiler_params=pltpu.CompilerParams(needs_layout_passes=False),
    scratch_types=[
        pltpu.VMEM_SHARED((NP, DH), jnp.float32),
        pltpu.VMEM_SHARED((NP, DH // 2), jnp.int32),
        pltpu.VMEM((2, 2 * GC, K), jnp.int32),
        pltpu.VMEM((2, GC, K), jnp.float32),
        pltpu.VMEM((2, K, DH), jnp.float32),
        pltpu.VMEM((2, K, DH // 2), jnp.int32),
        pltpu.SemaphoreType.DMA((2,)),
        pltpu.SemaphoreType.DMA((2,)),
        pltpu.SemaphoreType.DMA((2,)),
    ],
)
def _graph_conv_sc(emb2, er, wr5, out2, *scratch):
    _sc_body(emb2, er, wr5, out2, *scratch)


def kernel(user_emb, entity_emb, graph_indices, graph_values):
    all_embed = jnp.concatenate([user_emb, entity_emb], axis=0)
    all_embed = jnp.pad(all_embed, ((0, NP - N), (0, 0)))
    emb2 = jnp.stack([all_embed[:, :DH], all_embed[:, DH:]], axis=0)
    head = graph_indices[0]
    tail = graph_indices[1]
    pad = E_PAD - E
    tailr = jnp.pad(tail, (0, pad)).reshape(NS, NG, GC, K)
    headr = jnp.pad(head, (0, pad)).reshape(NS, NG, GC, K)
    wr = jnp.pad(graph_values, (0, pad)).reshape(NS, NG, GC, K)
    er = jnp.stack([tailr, headr], axis=3).reshape(NS, NG, 2 * GC, K)
    out2 = _graph_conv_sc(emb2, er, wr)
    acc = jnp.concatenate([out2[0, :N], out2[1, :N]], axis=1)
    return (acc[:N_USERS], acc[N_USERS:])


# speedup vs baseline: 1.0212x; 1.0212x over previous
import functools

import jax
import jax.numpy as jnp
from jax import lax
from jax.experimental import pallas as pl
from jax.experimental.pallas import tpu as pltpu
from jax.experimental.pallas import tpu_sc as plsc

N_USERS = 2000
N = 10000
NP = 10240
D = 128
E = 320000
N_HOPS = 3

NC = 2
NS = 16
DH = D // NC
RPT = NP // NS
K = 64
GC = 4
NG = 80
NCH = NG * GC
EPT = NCH * K
E_PAD = NS * EPT
NQ = RPT // K


def _splat(i):
    return jnp.full((16,), i, dtype=jnp.int32)


_GDN = lax.GatherDimensionNumbers(
    offset_dims=(), collapsed_slice_dims=(0,), start_index_map=(0,))


def _bcast_lane(v16, lane):
    return lax.gather(v16, _splat(lane)[:, None], _GDN, (1,),
                      mode=lax.GatherScatterMode.PROMISE_IN_BOUNDS)


def _sc_body(emb2, er, wr5, out2, snxt, sbf, ebuf, wbuf, gbuf, bbuf,
             se, sg, ss):
    c = lax.axis_index("c")
    s = lax.axis_index("s")
    row0 = s * RPT

    def fetch_group(g, slot):
        pltpu.async_copy(er.at[s, g], ebuf.at[slot], se.at[slot])
        pltpu.async_copy(wr5.at[s, g], wbuf.at[slot], se.at[slot])

    def wait_fetch(slot):
        pltpu.make_async_copy(er.at[s, 0], ebuf.at[slot], se.at[slot]).wait()
        pltpu.make_async_copy(wr5.at[s, 0], wbuf.at[slot], se.at[slot]).wait()

    def gather(slot, k, b):
        pltpu.async_copy(sbf.at[ebuf.at[slot, 2 * k]], bbuf.at[b], sg.at[b])

    def wait_gather(slot, k, b):
        pltpu.make_async_copy(sbf.at[ebuf.at[slot, 2 * k]], bbuf.at[b],
                              sg.at[b]).wait()

    def scatter(slot, k, b):
        pltpu.async_copy(gbuf.at[b], snxt.at[ebuf.at[slot, 2 * k + 1]],
                         ss.at[b], add=True)

    def drain_scatter(b):
        pltpu.make_async_copy(gbuf.at[b], snxt.at[ebuf.at[0, 1]],
                              ss.at[b]).wait()

    def scale(slot, k, b):
        def _scale32(g, _):
            for h in range(2):
                w16 = wbuf[slot, k, pl.ds(32 * g + 16 * h, 16)]
                bs = 32 * g + 16 * h
                for e16 in range(16):
                    wbc = _bcast_lane(w16, e16)
                    for q in range(DH // 32):
                        xw = bbuf[b, bs + e16, pl.ds(16 * q, 16)]
                        x = plsc.bitcast(xw, jnp.bfloat16)
                        a0, a1 = plsc.unpack(
                            x, format=plsc.PackFormat.INTERLEAVED)
                        gbuf[b, bs + e16, pl.ds(32 * q, 16)] = a0 * wbc
                        gbuf[b, bs + e16, pl.ds(32 * q + 16, 16)] = a1 * wbc
            return 0

        lax.fori_loop(0, K // 32, _scale32, 0)

    def pack_rows(nrows):
        def _prow(i, _):
            for h in range(DH // 32):
                a = gbuf[0, i, pl.ds(32 * h, 16)]
                b = gbuf[0, i, pl.ds(32 * h + 16, 16)]
                ab = plsc.pack(a, b, format=plsc.PackFormat.INTERLEAVED)
                bbuf[0, i, pl.ds(16 * h, 16)] = plsc.bitcast(ab, jnp.int32)
            return 0

        lax.fori_loop(0, nrows, _prow, 0)

    for q in range(NQ):
        sl = pl.ds(row0 + K * q, K)
        pltpu.sync_copy(emb2.at[c, sl], gbuf.at[0])
        pack_rows(K)
        pltpu.sync_copy(bbuf.at[0], sbf.at[sl])
    plsc.subcore_barrier()

    for hop in range(N_HOPS):

        def _zrow(i, _):
            for q in range(DH // 16):
                gbuf[0, i, pl.ds(16 * q, 16)] = jnp.zeros((16,), jnp.float32)
            return 0

        lax.fori_loop(0, K, _zrow, 0)
        for q in range(NQ):
            pltpu.sync_copy(gbuf.at[0], snxt.at[pl.ds(row0 + K * q, K)])
        plsc.subcore_barrier()

        def process_group(g, slot):
            wait_fetch(slot)
            for k in range(GC):
                @pl.when(g > 0)
                def _():
                    drain_scatter(k)
                gather(slot, k, k)

            @pl.when(g < NG - 1)
            def _():
                fetch_group(g + 1, 1 - slot)

            for k in range(GC):
                wait_gather(slot, k, k)
                scale(slot, k, k)
                scatter(slot, k, k)

        fetch_group(0, 0)

        @pl.loop(0, NG, step=2)
        def _pair(g):
            process_group(g, 0)
            process_group(g + 1, 1)

        for k in range(GC):
            drain_scatter(k)
        plsc.subcore_barrier()

        for q in range(NQ):
            sl = pl.ds(row0 + K * q, K)
            pltpu.sync_copy(snxt.at[sl], gbuf.at[0])
            if hop == 0:
                pltpu.sync_copy(emb2.at[c, sl], gbuf.at[1])
            else:
                pltpu.sync_copy(out2.at[c, sl], gbuf.at[1])

            def _acc(i, _):
                for q2 in range(DH // 16):
                    ksl = pl.ds(16 * q2, 16)
                    gbuf[1, i, ksl] = gbuf[1, i, ksl] + gbuf[0, i, ksl]
                return 0

            lax.fori_loop(0, K, _acc, 0)
            pltpu.sync_copy(gbuf.at[1], out2.at[c, sl])
            if hop < N_HOPS - 1:
                pack_rows(K)
                pltpu.sync_copy(bbuf.at[0], sbf.at[sl])
        plsc.subcore_barrier()


@functools.partial(
    pl.kernel,
    out_type=jax.ShapeDtypeStruct((NC, NP, DH), jnp.float32),
    mesh=plsc.VectorSubcoreMesh(core_axis_name="c", subcore_axis_name="s"),
    compiler_params=pltpu.CompilerParams(needs_layout_passes=False),
    scratch_types=[
        pltpu.VMEM_SHARED((NP, DH), jnp.float32),
        pltpu.VMEM_SHARED((NP, DH // 2), jnp.int32),
        pltpu.VMEM((2, 2 * GC, K), jnp.int32),
        pltpu.VMEM((2, GC, K), jnp.float32),
        pltpu.VMEM((4, K, DH), jnp.float32),
        pltpu.VMEM((4, K, DH // 2), jnp.int32),
        pltpu.SemaphoreType.DMA((2,)),
        pltpu.SemaphoreType.DMA((4,)),
        pltpu.SemaphoreType.DMA((4,)),
    ],
)
def _graph_conv_sc(emb2, er, wr5, out2, *scratch):
    _sc_body(emb2, er, wr5, out2, *scratch)


def kernel(user_emb, entity_emb, graph_indices, graph_values):
    all_embed = jnp.concatenate([user_emb, entity_emb], axis=0)
    all_embed = jnp.pad(all_embed, ((0, NP - N), (0, 0)))
    emb2 = jnp.stack([all_embed[:, :DH], all_embed[:, DH:]], axis=0)
    head = graph_indices[0]
    tail = graph_indices[1]
    pad = E_PAD - E
    tailr = jnp.pad(tail, (0, pad)).reshape(NS, NG, GC, K)
    headr = jnp.pad(head, (0, pad)).reshape(NS, NG, GC, K)
    wr = jnp.pad(graph_values, (0, pad)).reshape(NS, NG, GC, K)
    er = jnp.stack([tailr, headr], axis=3).reshape(NS, NG, 2 * GC, K)
    out2 = _graph_conv_sc(emb2, er, wr)
    acc = jnp.concatenate([out2[0, :N], out2[1, :N]], axis=1)
    return (acc[:N_USERS], acc[N_USERS:])


# speedup vs baseline: 1.0909x; 1.0682x over previous
import functools

import jax
import jax.numpy as jnp
from jax import lax
from jax.experimental import pallas as pl
from jax.experimental.pallas import tpu as pltpu
from jax.experimental.pallas import tpu_sc as plsc

N_USERS = 2000
N = 10000
NP = 10240
D = 128
E = 320000
N_HOPS = 3

NC = 2
NS = 16
DH = D // NC
RPT = NP // NS
K = 64
GC = 4
NG = 80
NCH = NG * GC
EPT = NCH * K
E_PAD = NS * EPT
NQ = RPT // K


def _splat(i):
    return jnp.full((16,), i, dtype=jnp.int32)


_GDN = lax.GatherDimensionNumbers(
    offset_dims=(), collapsed_slice_dims=(0,), start_index_map=(0,))


def _bcast_lane(v16, lane):
    return lax.gather(v16, _splat(lane)[:, None], _GDN, (1,),
                      mode=lax.GatherScatterMode.PROMISE_IN_BOUNDS)


def _sc_body(emb2, er, wr5, out2, snxt, sbf, ebuf, wbuf, gbuf, bbuf,
             se, sg, ss):
    c = lax.axis_index("c")
    s = lax.axis_index("s")
    row0 = s * RPT

    def fetch_group(g, slot):
        pltpu.async_copy(er.at[s, g], ebuf.at[slot], se.at[slot])
        pltpu.async_copy(wr5.at[s, g], wbuf.at[slot], se.at[slot])

    def wait_fetch(slot):
        pltpu.make_async_copy(er.at[s, 0], ebuf.at[slot], se.at[slot]).wait()
        pltpu.make_async_copy(wr5.at[s, 0], wbuf.at[slot], se.at[slot]).wait()

    def gather(slot, k, b):
        pltpu.async_copy(sbf.at[ebuf.at[slot, 2 * k]], bbuf.at[b], sg.at[b])

    def wait_gather(slot, k, b):
        pltpu.make_async_copy(sbf.at[ebuf.at[slot, 2 * k]], bbuf.at[b],
                              sg.at[b]).wait()

    def scatter(slot, k, b):
        pltpu.async_copy(gbuf.at[b], snxt.at[ebuf.at[slot, 2 * k + 1]],
                         ss.at[b], add=True)

    def drain_scatter(b):
        pltpu.make_async_copy(gbuf.at[b], snxt.at[ebuf.at[0, 1]],
                              ss.at[b]).wait()

    def scale(slot, k, b):
        def _scale32(g, _):
            for h in range(2):
                w16 = wbuf[slot, k, pl.ds(32 * g + 16 * h, 16)]
                bs = 32 * g + 16 * h
                for e16 in range(16):
                    wbc = _bcast_lane(w16, e16)
                    for q in range(DH // 32):
                        xw = bbuf[b, bs + e16, pl.ds(16 * q, 16)]
                        x = plsc.bitcast(xw, jnp.bfloat16)
                        a0, a1 = plsc.unpack(
                            x, format=plsc.PackFormat.INTERLEAVED)
                        gbuf[b, bs + e16, pl.ds(32 * q, 16)] = a0 * wbc
                        gbuf[b, bs + e16, pl.ds(32 * q + 16, 16)] = a1 * wbc
            return 0

        lax.fori_loop(0, K // 32, _scale32, 0)

    def pack_rows(src_b, dst_b):
        def _prow(i, _):
            for h in range(DH // 32):
                a = gbuf[src_b, i, pl.ds(32 * h, 16)]
                b = gbuf[src_b, i, pl.ds(32 * h + 16, 16)]
                ab = plsc.pack(a, b, format=plsc.PackFormat.INTERLEAVED)
                bbuf[dst_b, i, pl.ds(16 * h, 16)] = plsc.bitcast(ab, jnp.int32)
            return 0

        lax.fori_loop(0, K, _prow, 0)

    for q in range(NQ):
        sl = pl.ds(row0 + K * q, K)
        pltpu.sync_copy(emb2.at[c, sl], gbuf.at[0])
        pack_rows(0, 0)
        pltpu.sync_copy(bbuf.at[0], sbf.at[sl])
    plsc.subcore_barrier()

    for hop in range(N_HOPS):

        def _zrow(i, _):
            for q in range(DH // 16):
                gbuf[0, i, pl.ds(16 * q, 16)] = jnp.zeros((16,), jnp.float32)
            return 0

        lax.fori_loop(0, K, _zrow, 0)
        for q in range(NQ):
            pltpu.async_copy(gbuf.at[0], snxt.at[pl.ds(row0 + K * q, K)],
                             sg.at[0])
        for q in range(NQ):
            pltpu.make_async_copy(gbuf.at[0], snxt.at[pl.ds(row0, K)],
                                  sg.at[0]).wait()
        plsc.subcore_barrier()

        def process_group(g, slot):
            wait_fetch(slot)
            for k in range(GC):
                @pl.when(g > 0)
                def _():
                    drain_scatter(k)
                gather(slot, k, k)

            @pl.when(g < NG - 1)
            def _():
                fetch_group(g + 1, 1 - slot)

            for k in range(GC):
                wait_gather(slot, k, k)
                scale(slot, k, k)
                scatter(slot, k, k)

        fetch_group(0, 0)

        @pl.loop(0, NG, step=2)
        def _pair(g):
            process_group(g, 0)
            process_group(g + 1, 1)

        for k in range(GC):
            drain_scatter(k)
        plsc.subcore_barrier()

        def rd_issue(q):
            p = q % 2
            sl = pl.ds(row0 + K * q, K)
            pltpu.async_copy(snxt.at[sl], gbuf.at[2 * p], sg.at[2 * p])
            prev = emb2.at[c, sl] if hop == 0 else out2.at[c, sl]
            pltpu.async_copy(prev, gbuf.at[2 * p + 1], sg.at[2 * p + 1])

        def rd_wait(q):
            p = q % 2
            sl = pl.ds(row0 + K * q, K)
            pltpu.make_async_copy(snxt.at[sl], gbuf.at[2 * p],
                                  sg.at[2 * p]).wait()
            prev = emb2.at[c, sl] if hop == 0 else out2.at[c, sl]
            pltpu.make_async_copy(prev, gbuf.at[2 * p + 1],
                                  sg.at[2 * p + 1]).wait()

        def wr_drain(q):
            p = q % 2
            sl = pl.ds(row0 + K * q, K)
            pltpu.make_async_copy(gbuf.at[2 * p + 1], out2.at[c, sl],
                                  ss.at[p]).wait()
            if hop < N_HOPS - 1:
                pltpu.make_async_copy(bbuf.at[p], sbf.at[sl],
                                      se.at[p]).wait()

        rd_issue(0)
        for q in range(NQ):
            p = q % 2
            sl = pl.ds(row0 + K * q, K)
            rd_wait(q)
            if q + 1 < NQ:
                if q >= 1:
                    wr_drain(q - 1)
                rd_issue(q + 1)

            def _acc(i, _):
                for q2 in range(DH // 16):
                    ksl = pl.ds(16 * q2, 16)
                    gbuf[2 * p + 1, i, ksl] = (gbuf[2 * p + 1, i, ksl]
                                               + gbuf[2 * p, i, ksl])
                return 0

            lax.fori_loop(0, K, _acc, 0)
            pltpu.async_copy(gbuf.at[2 * p + 1], out2.at[c, sl], ss.at[p])
            if hop < N_HOPS - 1:
                pack_rows(2 * p, p)
                pltpu.async_copy(bbuf.at[p], sbf.at[sl], se.at[p])
        wr_drain(NQ - 2)
        wr_drain(NQ - 1)
        plsc.subcore_barrier()


@functools.partial(
    pl.kernel,
    out_type=jax.ShapeDtypeStruct((NC, NP, DH), jnp.float32),
    mesh=plsc.VectorSubcoreMesh(core_axis_name="c", subcore_axis_name="s"),
    compiler_params=pltpu.CompilerParams(needs_layout_passes=False),
    scratch_types=[
        pltpu.VMEM_SHARED((NP, DH), jnp.float32),
        pltpu.VMEM_SHARED((NP, DH // 2), jnp.int32),
        pltpu.VMEM((2, 2 * GC, K), jnp.int32),
        pltpu.VMEM((2, GC, K), jnp.float32),
        pltpu.VMEM((4, K, DH), jnp.float32),
        pltpu.VMEM((4, K, DH // 2), jnp.int32),
        pltpu.SemaphoreType.DMA((2,)),
        pltpu.SemaphoreType.DMA((4,)),
        pltpu.SemaphoreType.DMA((4,)),
    ],
)
def _graph_conv_sc(emb2, er, wr5, out2, *scratch):
    _sc_body(emb2, er, wr5, out2, *scratch)


def kernel(user_emb, entity_emb, graph_indices, graph_values):
    all_embed = jnp.concatenate([user_emb, entity_emb], axis=0)
    all_embed = jnp.pad(all_embed, ((0, NP - N), (0, 0)))
    emb2 = jnp.stack([all_embed[:, :DH], all_embed[:, DH:]], axis=0)
    head = graph_indices[0]
    tail = graph_indices[1]
    pad = E_PAD - E
    tailr = jnp.pad(tail, (0, pad)).reshape(NS, NG, GC, K)
    headr = jnp.pad(head, (0, pad)).reshape(NS, NG, GC, K)
    wr = jnp.pad(graph_values, (0, pad)).reshape(NS, NG, GC, K)
    er = jnp.stack([tailr, headr], axis=3).reshape(NS, NG, 2 * GC, K)
    out2 = _graph_conv_sc(emb2, er, wr)
    acc = jnp.concatenate([out2[0, :N], out2[1, :N]], axis=1)
    return (acc[:N_USERS], acc[N_USERS:])
